# baseline (speedup 1.0000x reference)
import functools

import jax
import jax.numpy as jnp
from jax import lax
from jax.experimental import pallas as pl
from jax.experimental.pallas import tpu as pltpu
from jax.experimental.pallas import tpu_sc as plsc

DIM = 64
LANES = 16
CHUNK = 128

_ROUND_MAGIC = 8388608.0


def _sc_gather(arr, idx_flat, n_total, n_per_worker):
    n_chunks = n_per_worker // CHUNK

    mesh = plsc.VectorSubcoreMesh(core_axis_name="c", subcore_axis_name="s")

    @functools.partial(
        pl.kernel,
        mesh=mesh,
        out_type=[
            jax.ShapeDtypeStruct((n_total, DIM), jnp.float32),
            jax.ShapeDtypeStruct((n_total,), jnp.float32),
        ],
        scratch_types=[
            pltpu.VMEM((n_per_worker,), jnp.float32),
            pltpu.VMEM((n_per_worker,), jnp.int32),
            pltpu.VMEM((n_per_worker,), jnp.float32),
            pltpu.VMEM((CHUNK, DIM), jnp.float32),
            pltpu.SemaphoreType.DMA,
        ],
        compiler_params=pltpu.CompilerParams(use_tc_tiling_on_sc=False),
    )
    def k(arr_hbm, idxf_hbm, out_hbm, res_hbm, idxf_v, idx_v, res_v, rows_v, sem):
        nc = 2
        wid = lax.axis_index("s") * nc + lax.axis_index("c")
        base = wid * n_per_worker

        pltpu.sync_copy(idxf_hbm.at[pl.ds(base, n_per_worker)], idxf_v)

        def round_body(i, carry):
            x = idxf_v[pl.ds(i * LANES, LANES)]
            r = (x + _ROUND_MAGIC) - _ROUND_MAGIC
            res_v[pl.ds(i * LANES, LANES)] = x - r
            idx_v[pl.ds(i * LANES, LANES)] = r.astype(jnp.int32)
            return carry

        lax.fori_loop(0, n_per_worker // LANES, round_body, 0)

        pltpu.sync_copy(res_v, res_hbm.at[pl.ds(base, n_per_worker)])

        def gather_body(j, carry):
            idx_ref = idx_v.at[pl.ds(j * CHUNK, CHUNK)]
            pltpu.async_copy(arr_hbm.at[idx_ref], rows_v, sem).wait()
            pltpu.sync_copy(
                rows_v, out_hbm.at[pl.ds(base + j * CHUNK, CHUNK)]
            )
            return carry

        lax.fori_loop(0, n_chunks, gather_body, 0)

    return k(arr, idx_flat)


def kernel(arr, index):
    batch, fields = index.shape
    n_total = batch * fields
    n_per_worker = n_total // 32
    idx_flat = index.reshape(n_total)
    rows, res = _sc_gather(arr, idx_flat, n_total, n_per_worker)
    return rows.reshape(batch, fields, DIM), res.reshape(batch, fields)


# speedup vs baseline: 1.0746x; 1.0746x over previous
import functools

import jax
import jax.numpy as jnp
from jax import lax
from jax.experimental import pallas as pl
from jax.experimental.pallas import tpu as pltpu
from jax.experimental.pallas import tpu_sc as plsc

DIM = 64
LANES = 16
CHUNK = 128
NBUF = 4

_ROUND_MAGIC = 8388608.0


def _sc_gather(arr, idx_flat, n_total, n_per_worker):
    n_chunks = n_per_worker // CHUNK
    lead = NBUF // 2
    assert (n_chunks - 2 * lead) % NBUF == 0

    mesh = plsc.VectorSubcoreMesh(core_axis_name="c", subcore_axis_name="s")

    @functools.partial(
        pl.kernel,
        mesh=mesh,
        out_type=[
            jax.ShapeDtypeStruct((n_total, DIM), jnp.float32),
            jax.ShapeDtypeStruct((n_total,), jnp.float32),
        ],
        scratch_types=[
            pltpu.VMEM((n_per_worker,), jnp.float32),
            pltpu.VMEM((n_per_worker,), jnp.int32),
            pltpu.VMEM((n_per_worker,), jnp.float32),
            [pltpu.VMEM((CHUNK, DIM), jnp.float32)] * NBUF,
            [pltpu.SemaphoreType.DMA] * NBUF,
            [pltpu.SemaphoreType.DMA] * NBUF,
            pltpu.SemaphoreType.DMA,
        ],
        compiler_params=pltpu.CompilerParams(use_tc_tiling_on_sc=False),
    )
    def k(arr_hbm, idxf_hbm, out_hbm, res_hbm, idxf_v, idx_v, res_v,
          rows_v, gsems, ssems, rsem):
        nc = 2
        wid = lax.axis_index("s") * nc + lax.axis_index("c")
        base = wid * n_per_worker

        pltpu.sync_copy(idxf_hbm.at[pl.ds(base, n_per_worker)], idxf_v)

        def round_body(i, carry):
            x = idxf_v[pl.ds(i * LANES, LANES)]
            r = (x + _ROUND_MAGIC) - _ROUND_MAGIC
            res_v[pl.ds(i * LANES, LANES)] = x - r
            idx_v[pl.ds(i * LANES, LANES)] = r.astype(jnp.int32)
            return carry

        lax.fori_loop(0, n_per_worker // LANES, round_body, 0)

        res_copy = pltpu.async_copy(
            res_v, res_hbm.at[pl.ds(base, n_per_worker)], rsem
        )

        def start_gather(j, b):
            idx_ref = idx_v.at[pl.ds(j * CHUNK, CHUNK)]
            pltpu.async_copy(arr_hbm.at[idx_ref], rows_v[b], gsems[b])

        def wait_gather(b):
            idx_ref = idx_v.at[pl.ds(0, CHUNK)]
            pltpu.make_async_copy(arr_hbm.at[idx_ref], rows_v[b], gsems[b]).wait()

        def start_store(j, b):
            pltpu.async_copy(
                rows_v[b], out_hbm.at[pl.ds(base + j * CHUNK, CHUNK)], ssems[b]
            )

        def wait_store(b):
            pltpu.make_async_copy(
                rows_v[b], out_hbm.at[pl.ds(base, CHUNK)], ssems[b]
            ).wait()

        for j in range(lead):
            start_gather(j, j % NBUF)
        for j in range(lead):
            start_gather(j + lead, (j + lead) % NBUF)
            wait_gather(j % NBUF)
            start_store(j, j % NBUF)

        def steady(jo, carry):
            for bi in range(NBUF):
                j = lead + jo * NBUF + bi
                b = (lead + bi) % NBUF
                b2 = (b + lead) % NBUF
                wait_store(b2)
                start_gather(j + lead, b2)
                wait_gather(b)
                start_store(j, b)
            return carry

        lax.fori_loop(0, (n_chunks - 2 * lead) // NBUF, steady, 0)

        for j in range(n_chunks - lead, n_chunks):
            b = j % NBUF
            wait_gather(b)
            start_store(j, b)
        for j in range(n_chunks - NBUF, n_chunks):
            wait_store(j % NBUF)
        res_copy.wait()

    return k(arr, idx_flat)


def kernel(arr, index):
    batch, fields = index.shape
    n_total = batch * fields
    n_per_worker = n_total // 32
    idx_flat = index.reshape(n_total)
    rows, res = _sc_gather(arr, idx_flat, n_total, n_per_worker)
    return rows.reshape(batch, fields, DIM), res.reshape(batch, fields)


# speedup vs baseline: 1.0775x; 1.0027x over previous
import functools

import jax
import jax.numpy as jnp
from jax import lax
from jax.experimental import pallas as pl
from jax.experimental.pallas import tpu as pltpu
from jax.experimental.pallas import tpu_sc as plsc

DIM = 64
LANES = 16
CHUNK = 256
NBUF = 4

_ROUND_MAGIC = 8388608.0


def _sc_gather(arr, idx_flat, n_total, n_per_worker):
    n_chunks = n_per_worker // CHUNK
    lead = NBUF // 2
    assert (n_chunks - 2 * lead) % NBUF == 0

    mesh = plsc.VectorSubcoreMesh(core_axis_name="c", subcore_axis_name="s")

    @functools.partial(
        pl.kernel,
        mesh=mesh,
        out_type=[
            jax.ShapeDtypeStruct((n_total, DIM), jnp.float32),
            jax.ShapeDtypeStruct((n_total,), jnp.float32),
        ],
        scratch_types=[
            pltpu.VMEM((n_per_worker,), jnp.float32),
            pltpu.VMEM((n_per_worker,), jnp.int32),
            pltpu.VMEM((n_per_worker,), jnp.float32),
            [pltpu.VMEM((CHUNK, DIM), jnp.float32)] * NBUF,
            [pltpu.SemaphoreType.DMA] * NBUF,
            [pltpu.SemaphoreType.DMA] * NBUF,
            pltpu.SemaphoreType.DMA,
        ],
        compiler_params=pltpu.CompilerParams(use_tc_tiling_on_sc=False),
    )
    def k(arr_hbm, idxf_hbm, out_hbm, res_hbm, idxf_v, idx_v, res_v,
          rows_v, gsems, ssems, rsem):
        nc = 2
        wid = lax.axis_index("s") * nc + lax.axis_index("c")
        base = wid * n_per_worker

        pltpu.sync_copy(idxf_hbm.at[pl.ds(base, n_per_worker)], idxf_v)

        def round_body(i, carry):
            x = idxf_v[pl.ds(i * LANES, LANES)]
            r = (x + _ROUND_MAGIC) - _ROUND_MAGIC
            res_v[pl.ds(i * LANES, LANES)] = x - r
            idx_v[pl.ds(i * LANES, LANES)] = r.astype(jnp.int32)
            return carry

        lax.fori_loop(0, n_per_worker // LANES, round_body, 0)

        res_copy = pltpu.async_copy(
            res_v, res_hbm.at[pl.ds(base, n_per_worker)], rsem
        )

        def start_gather(j, b):
            idx_ref = idx_v.at[pl.ds(j * CHUNK, CHUNK)]
            pltpu.async_copy(arr_hbm.at[idx_ref], rows_v[b], gsems[b])

        def wait_gather(b):
            idx_ref = idx_v.at[pl.ds(0, CHUNK)]
            pltpu.make_async_copy(arr_hbm.at[idx_ref], rows_v[b], gsems[b]).wait()

        def start_store(j, b):
            pltpu.async_copy(
                rows_v[b], out_hbm.at[pl.ds(base + j * CHUNK, CHUNK)], ssems[b]
            )

        def wait_store(b):
            pltpu.make_async_copy(
                rows_v[b], out_hbm.at[pl.ds(base, CHUNK)], ssems[b]
            ).wait()

        for j in range(lead):
            start_gather(j, j % NBUF)
        for j in range(lead):
            start_gather(j + lead, (j + lead) % NBUF)
            wait_gather(j % NBUF)
            start_store(j, j % NBUF)

        def steady(jo, carry):
            for bi in range(NBUF):
                j = lead + jo * NBUF + bi
                b = (lead + bi) % NBUF
                b2 = (b + lead) % NBUF
                wait_store(b2)
                start_gather(j + lead, b2)
                wait_gather(b)
                start_store(j, b)
            return carry

        lax.fori_loop(0, (n_chunks - 2 * lead) // NBUF, steady, 0)

        for j in range(n_chunks - lead, n_chunks):
            b = j % NBUF
            wait_gather(b)
            start_store(j, b)
        for j in range(n_chunks - NBUF, n_chunks):
            wait_store(j % NBUF)
        res_copy.wait()

    return k(arr, idx_flat)


def kernel(arr, index):
    batch, fields = index.shape
    n_total = batch * fields
    n_per_worker = n_total // 32
    idx_flat = index.reshape(n_total)
    rows, res = _sc_gather(arr, idx_flat, n_total, n_per_worker)
    return rows.reshape(batch, fields, DIM), res.reshape(batch, fields)
